# baseline (speedup 1.0000x reference)
import functools

import jax
import jax.numpy as jnp
from jax.experimental import pallas as pl
from jax.experimental.pallas import tpu as pltpu

ALPHA = 0.05
BLK = 256


def _router_kernel(x_ref, wg_ref, bg_ref, idx_ref, p_ref, cnt_ref, aux_ref):
    x = x_ref[...]
    wg = wg_ref[...]
    e = wg.shape[0]
    logits = jax.lax.dot_general(
        x, wg, (((1,), (1,)), ((), ())), preferred_element_type=jnp.float32
    ) + bg_ref[...]
    lmax = jnp.max(logits, axis=1, keepdims=True)
    el = jnp.exp(logits - lmax)
    sum_el = jnp.sum(el, axis=1, keepdims=True)
    probs = el / sum_el
    iota = jax.lax.broadcasted_iota(jnp.int32, logits.shape, 1)
    idx = jnp.min(jnp.where(logits == lmax, iota, e), axis=1, keepdims=True)
    idx_ref[...] = idx
    p_ref[...] = 1.0 / sum_el
    onehot = (iota == idx).astype(jnp.float32)
    cnt = jnp.sum(onehot, axis=0, keepdims=True)
    cnt_ref[...] = cnt
    ce = jnp.mean(probs, axis=0, keepdims=True)
    me = cnt / x.shape[0]
    aux_ref[...] = jnp.reshape((ALPHA * e) * jnp.sum(me * ce), (1, 1))


def _ffn_kernel(be_ref, fl_ref, xs_ref, ps_ref, wu_ref, wv_ref, wd_ref, ys_ref):
    b = pl.program_id(0)

    @pl.when(fl_ref[b] == 1)
    def _():
        xb = xs_ref[...]
        u = jax.lax.dot_general(
            xb, wu_ref[0], (((1,), (1,)), ((), ())),
            preferred_element_type=jnp.float32)
        v = jax.lax.dot_general(
            xb, wv_ref[0], (((1,), (1,)), ((), ())),
            preferred_element_type=jnp.float32)
        h = ((u * jax.nn.sigmoid(u)) * v).astype(jnp.bfloat16)
        y = jax.lax.dot_general(
            h, wd_ref[0], (((1,), (1,)), ((), ())),
            preferred_element_type=jnp.float32)
        ys_ref[...] = y * ps_ref[...]

    @pl.when(fl_ref[b] == 0)
    def _():
        ys_ref[...] = jnp.zeros_like(ys_ref)


@functools.partial(jax.jit, static_argnames=())
def kernel(x, Wg, bg, Wu, Wv, Wd):
    n, d = x.shape
    e, h, _ = Wu.shape
    nb = n // BLK + e
    s = nb * BLK

    idx2, p2, cnt2, aux2 = pl.pallas_call(
        _router_kernel,
        out_shape=[
            jax.ShapeDtypeStruct((n, 1), jnp.int32),
            jax.ShapeDtypeStruct((n, 1), jnp.float32),
            jax.ShapeDtypeStruct((1, e), jnp.float32),
            jax.ShapeDtypeStruct((1, 1), jnp.float32),
        ],
    )(x, Wg, bg.reshape(1, e))
    top1_idx = idx2[:, 0]
    top1_p = p2[:, 0]
    counts = cnt2[0].astype(jnp.int32)
    aux = aux2.reshape(())

    nblk_e = (counts + BLK - 1) // BLK
    cum_nblk = jnp.cumsum(nblk_e)
    pstart = (cum_nblk - nblk_e) * BLK
    nba = cum_nblk[-1]

    onehot = jax.nn.one_hot(top1_idx, e, dtype=jnp.int32)
    rank = jnp.take_along_axis(jnp.cumsum(onehot, axis=0),
                               top1_idx[:, None], axis=1)[:, 0] - 1
    dest = pstart[top1_idx] + rank

    blk_ids = jnp.arange(nb, dtype=jnp.int32)
    be = jnp.searchsorted(cum_nblk, blk_ids, side="right").astype(jnp.int32)
    active = be < e
    last_e = jnp.searchsorted(cum_nblk, nba - 1, side="right").astype(jnp.int32)
    be_safe = jnp.where(active, be, last_e)
    flags = active.astype(jnp.int32)

    xs = jnp.zeros((s, d), jnp.bfloat16).at[dest].set(x.astype(jnp.bfloat16))
    ps = jnp.zeros((s, 1), jnp.float32).at[dest, 0].set(top1_p)

    grid_spec = pltpu.PrefetchScalarGridSpec(
        num_scalar_prefetch=2,
        grid=(nb,),
        in_specs=[
            pl.BlockSpec((BLK, d), lambda k, be, fl: (k, 0)),
            pl.BlockSpec((BLK, 1), lambda k, be, fl: (k, 0)),
            pl.BlockSpec((1, h, d), lambda k, be, fl: (be[k], 0, 0)),
            pl.BlockSpec((1, h, d), lambda k, be, fl: (be[k], 0, 0)),
            pl.BlockSpec((1, d, h), lambda k, be, fl: (be[k], 0, 0)),
        ],
        out_specs=pl.BlockSpec((BLK, d), lambda k, be, fl: (k, 0)),
    )
    ys = pl.pallas_call(
        _ffn_kernel,
        grid_spec=grid_spec,
        out_shape=jax.ShapeDtypeStruct((s, d), jnp.float32),
        compiler_params=pltpu.CompilerParams(
            dimension_semantics=("arbitrary",),
            vmem_limit_bytes=64 * 1024 * 1024,
        ),
    )(be_safe, flags, xs, ps,
      Wu.astype(jnp.bfloat16), Wv.astype(jnp.bfloat16), Wd.astype(jnp.bfloat16))

    y = jnp.take(ys, dest, axis=0)
    return y, aux


# speedup vs baseline: 1.0875x; 1.0875x over previous
import functools

import jax
import jax.numpy as jnp
from jax.experimental import pallas as pl
from jax.experimental.pallas import tpu as pltpu

ALPHA = 0.05
BLK = 512
KH = 2


def _router_kernel(x_ref, wg_ref, bg_ref, idx_ref, p_ref, cnt_ref, aux_ref):
    x = x_ref[...]
    wg = wg_ref[...]
    e = wg.shape[0]
    logits = jax.lax.dot_general(
        x, wg, (((1,), (1,)), ((), ())), preferred_element_type=jnp.float32
    ) + bg_ref[...]
    lmax = jnp.max(logits, axis=1, keepdims=True)
    el = jnp.exp(logits - lmax)
    sum_el = jnp.sum(el, axis=1, keepdims=True)
    probs = el / sum_el
    iota = jax.lax.broadcasted_iota(jnp.int32, logits.shape, 1)
    idx = jnp.min(jnp.where(logits == lmax, iota, e), axis=1, keepdims=True)
    idx_ref[...] = idx
    p_ref[...] = 1.0 / sum_el
    onehot = (iota == idx).astype(jnp.float32)
    cnt = jnp.sum(onehot, axis=0, keepdims=True)
    cnt_ref[...] = cnt
    ce = jnp.mean(probs, axis=0, keepdims=True)
    me = cnt / x.shape[0]
    aux_ref[...] = jnp.reshape((ALPHA * e) * jnp.sum(me * ce), (1, 1))


def _up_kernel(be_ref, fl_ref, xs_ref, wu_ref, wv_ref, hs_ref):
    b = pl.program_id(1)

    @pl.when(fl_ref[b] == 1)
    def _():
        xb = xs_ref[...]
        ut = jax.lax.dot_general(
            wu_ref[0], xb, (((1,), (1,)), ((), ())),
            preferred_element_type=jnp.float32)
        vt = jax.lax.dot_general(
            wv_ref[0], xb, (((1,), (1,)), ((), ())),
            preferred_element_type=jnp.float32)
        hs_ref[...] = ((ut * jax.nn.sigmoid(ut)) * vt).astype(jnp.bfloat16)


def _down_kernel(be_ref, fl_ref, hs_ref, ps_ref, wd_ref, ys_ref):
    b = pl.program_id(0)

    @pl.when(fl_ref[b] == 1)
    def _():
        ht_ = hs_ref[...].astype(jnp.float32)
        y = jax.lax.dot_general(
            ht_, wd_ref[0], (((0,), (1,)), ((), ())),
            preferred_element_type=jnp.float32)
        ys_ref[...] = y * ps_ref[...]

    @pl.when(fl_ref[b] == 0)
    def _():
        ys_ref[...] = jnp.zeros_like(ys_ref)


@functools.partial(jax.jit, static_argnames=())
def kernel(x, Wg, bg, Wu, Wv, Wd):
    n, d = x.shape
    e, h, _ = Wu.shape
    ht = h // KH
    nb = n // BLK + e
    s = nb * BLK

    idx2, p2, cnt2, aux2 = pl.pallas_call(
        _router_kernel,
        out_shape=[
            jax.ShapeDtypeStruct((n, 1), jnp.int32),
            jax.ShapeDtypeStruct((n, 1), jnp.float32),
            jax.ShapeDtypeStruct((1, e), jnp.float32),
            jax.ShapeDtypeStruct((1, 1), jnp.float32),
        ],
    )(x, Wg, bg.reshape(1, e))
    top1_idx = idx2[:, 0]
    top1_p = p2[:, 0]
    counts = cnt2[0].astype(jnp.int32)
    aux = aux2.reshape(())

    nblk_e = (counts + BLK - 1) // BLK
    cum_nblk = jnp.cumsum(nblk_e)
    pstart = (cum_nblk - nblk_e) * BLK
    nba = cum_nblk[-1]

    onehot = jax.nn.one_hot(top1_idx, e, dtype=jnp.int32)
    rank = jnp.take_along_axis(jnp.cumsum(onehot, axis=0),
                               top1_idx[:, None], axis=1)[:, 0] - 1
    dest = pstart[top1_idx] + rank

    blk_ids = jnp.arange(nb, dtype=jnp.int32)
    be = jnp.searchsorted(cum_nblk, blk_ids, side="right").astype(jnp.int32)
    active = be < e
    last_e = jnp.searchsorted(cum_nblk, nba - 1, side="right").astype(jnp.int32)
    be_safe = jnp.where(active, be, last_e)
    flags = active.astype(jnp.int32)

    xs = jnp.zeros((s, d), jnp.float32).at[dest].set(x)
    ps = jnp.zeros((s, 1), jnp.float32).at[dest, 0].set(top1_p)

    wu4 = Wu.reshape(e * KH, ht, d)
    wv4 = Wv.reshape(e * KH, ht, d)

    up_spec = pltpu.PrefetchScalarGridSpec(
        num_scalar_prefetch=2,
        grid=(KH, nb),
        in_specs=[
            pl.BlockSpec((BLK, d), lambda hi, b, be, fl: (b, 0)),
            pl.BlockSpec((1, ht, d), lambda hi, b, be, fl: (be[b] * KH + hi, 0, 0)),
            pl.BlockSpec((1, ht, d), lambda hi, b, be, fl: (be[b] * KH + hi, 0, 0)),
        ],
        out_specs=pl.BlockSpec((ht, BLK), lambda hi, b, be, fl: (hi, b)),
    )
    hs = pl.pallas_call(
        _up_kernel,
        grid_spec=up_spec,
        out_shape=jax.ShapeDtypeStruct((h, s), jnp.bfloat16),
        compiler_params=pltpu.CompilerParams(
            dimension_semantics=("arbitrary", "arbitrary"),
        ),
    )(be_safe, flags, xs, wu4, wv4)

    down_spec = pltpu.PrefetchScalarGridSpec(
        num_scalar_prefetch=2,
        grid=(nb,),
        in_specs=[
            pl.BlockSpec((h, BLK), lambda b, be, fl: (0, b)),
            pl.BlockSpec((BLK, 1), lambda b, be, fl: (b, 0)),
            pl.BlockSpec((1, d, h), lambda b, be, fl: (be[b], 0, 0)),
        ],
        out_specs=pl.BlockSpec((BLK, d), lambda b, be, fl: (b, 0)),
    )
    ys = pl.pallas_call(
        _down_kernel,
        grid_spec=down_spec,
        out_shape=jax.ShapeDtypeStruct((s, d), jnp.float32),
        compiler_params=pltpu.CompilerParams(
            dimension_semantics=("arbitrary",),
        ),
    )(be_safe, flags, hs, ps, Wd)

    y = jnp.take(ys, dest, axis=0)
    return y, aux


# speedup vs baseline: 1.1365x; 1.0451x over previous
import functools

import jax
import jax.numpy as jnp
from jax.experimental import pallas as pl
from jax.experimental.pallas import tpu as pltpu
from jax.experimental.pallas import tpu_sc as plsc

ALPHA = 0.05
BLK = 512
KH = 2


def _router_kernel(x_ref, wg_ref, bg_ref, idx_ref, p_ref, cnt_ref, aux_ref):
    x = x_ref[...]
    wg = wg_ref[...]
    e = wg.shape[0]
    logits = jax.lax.dot_general(
        x, wg, (((1,), (1,)), ((), ())), preferred_element_type=jnp.float32
    ) + bg_ref[...]
    lmax = jnp.max(logits, axis=1, keepdims=True)
    el = jnp.exp(logits - lmax)
    sum_el = jnp.sum(el, axis=1, keepdims=True)
    probs = el / sum_el
    iota = jax.lax.broadcasted_iota(jnp.int32, logits.shape, 1)
    idx = jnp.min(jnp.where(logits == lmax, iota, e), axis=1, keepdims=True)
    idx_ref[...] = idx
    p_ref[...] = 1.0 / sum_el
    onehot = (iota == idx).astype(jnp.float32)
    cnt = jnp.sum(onehot, axis=0, keepdims=True)
    cnt_ref[...] = cnt
    ce = jnp.mean(probs, axis=0, keepdims=True)
    me = cnt / x.shape[0]
    aux_ref[...] = jnp.reshape((ALPHA * e) * jnp.sum(me * ce), (1, 1))


def _up_kernel(be_ref, fl_ref, xs_ref, wu_ref, wv_ref, hs_ref):
    b = pl.program_id(1)

    @pl.when(fl_ref[b] == 1)
    def _():
        xb = xs_ref[...]
        ut = jax.lax.dot_general(
            wu_ref[0], xb, (((1,), (1,)), ((), ())),
            preferred_element_type=jnp.float32)
        vt = jax.lax.dot_general(
            wv_ref[0], xb, (((1,), (1,)), ((), ())),
            preferred_element_type=jnp.float32)
        hs_ref[...] = ((ut * jax.nn.sigmoid(ut)) * vt).astype(jnp.bfloat16)


def _down_kernel(be_ref, fl_ref, hs_ref, ps_ref, wd_ref, ys_ref):
    b = pl.program_id(0)

    @pl.when(fl_ref[b] == 1)
    def _():
        ht_ = hs_ref[...].astype(jnp.float32)
        y = jax.lax.dot_general(
            ht_, wd_ref[0], (((0,), (1,)), ((), ())),
            preferred_element_type=jnp.float32)
        ys_ref[...] = y * ps_ref[...]

    @pl.when(fl_ref[b] == 0)
    def _():
        ys_ref[...] = jnp.zeros_like(ys_ref)


@functools.partial(jax.jit, static_argnames=())
def kernel(x, Wg, bg, Wu, Wv, Wd):
    n, d = x.shape
    e, h, _ = Wu.shape
    ht = h // KH
    nb = n // BLK + e
    s = nb * BLK

    idx2, p2, cnt2, aux2 = pl.pallas_call(
        _router_kernel,
        out_shape=[
            jax.ShapeDtypeStruct((n, 1), jnp.int32),
            jax.ShapeDtypeStruct((n, 1), jnp.float32),
            jax.ShapeDtypeStruct((1, e), jnp.float32),
            jax.ShapeDtypeStruct((1, 1), jnp.float32),
        ],
    )(x, Wg, bg.reshape(1, e))
    top1_idx = idx2[:, 0]
    top1_p = p2[:, 0]
    counts = cnt2[0].astype(jnp.int32)
    aux = aux2.reshape(())

    nblk_e = (counts + BLK - 1) // BLK
    cum_nblk = jnp.cumsum(nblk_e)
    pstart = (cum_nblk - nblk_e) * BLK
    nba = cum_nblk[-1]

    onehot = jax.nn.one_hot(top1_idx, e, dtype=jnp.int32)
    rank = jnp.take_along_axis(jnp.cumsum(onehot, axis=0),
                               top1_idx[:, None], axis=1)[:, 0] - 1
    dest = pstart[top1_idx] + rank

    blk_ids = jnp.arange(nb, dtype=jnp.int32)
    be = jnp.searchsorted(cum_nblk, blk_ids, side="right").astype(jnp.int32)
    active = be < e
    last_e = jnp.searchsorted(cum_nblk, nba - 1, side="right").astype(jnp.int32)
    be_safe = jnp.where(active, be, last_e)
    flags = active.astype(jnp.int32)

    xs = jnp.zeros((s, d), jnp.float32).at[dest].set(x)
    ps = jnp.zeros((s, 1), jnp.float32).at[dest, 0].set(top1_p)

    wu4 = Wu.reshape(e * KH, ht, d)
    wv4 = Wv.reshape(e * KH, ht, d)

    up_spec = pltpu.PrefetchScalarGridSpec(
        num_scalar_prefetch=2,
        grid=(KH, nb),
        in_specs=[
            pl.BlockSpec((BLK, d), lambda hi, b, be, fl: (b, 0)),
            pl.BlockSpec((1, ht, d), lambda hi, b, be, fl: (be[b] * KH + hi, 0, 0)),
            pl.BlockSpec((1, ht, d), lambda hi, b, be, fl: (be[b] * KH + hi, 0, 0)),
        ],
        out_specs=pl.BlockSpec((ht, BLK), lambda hi, b, be, fl: (hi, b)),
    )
    hs = pl.pallas_call(
        _up_kernel,
        grid_spec=up_spec,
        out_shape=jax.ShapeDtypeStruct((h, s), jnp.bfloat16),
        compiler_params=pltpu.CompilerParams(
            dimension_semantics=("arbitrary", "arbitrary"),
        ),
    )(be_safe, flags, xs, wu4, wv4)

    down_spec = pltpu.PrefetchScalarGridSpec(
        num_scalar_prefetch=2,
        grid=(nb,),
        in_specs=[
            pl.BlockSpec((h, BLK), lambda b, be, fl: (0, b)),
            pl.BlockSpec((BLK, 1), lambda b, be, fl: (b, 0)),
            pl.BlockSpec((1, d, h), lambda b, be, fl: (be[b], 0, 0)),
        ],
        out_specs=pl.BlockSpec((BLK, d), lambda b, be, fl: (b, 0)),
    )
    ys = pl.pallas_call(
        _down_kernel,
        grid_spec=down_spec,
        out_shape=jax.ShapeDtypeStruct((s, d), jnp.float32),
        compiler_params=pltpu.CompilerParams(
            dimension_semantics=("arbitrary",),
        ),
    )(be_safe, flags, hs, ps, Wd)

    y = _sc_combine(ys, dest, n, d)
    return y, aux


def _sc_combine(ys, dest, n, d):
    info = plsc.get_sparse_core_info()
    nw = info.num_cores * info.num_subcores
    bpw = n // nw
    chunk = 64
    nchunk = bpw // chunk
    mesh = plsc.VectorSubcoreMesh(core_axis_name="c", subcore_axis_name="s")

    @functools.partial(
        pl.kernel, mesh=mesh,
        out_type=jax.ShapeDtypeStruct((n, d), jnp.float32),
        scratch_types=[
            pltpu.VMEM((chunk,), jnp.int32),
            pltpu.VMEM((chunk, d), jnp.float32),
            pltpu.SemaphoreType.DMA,
        ],
    )
    def k(ys_hbm, dest_hbm, out_hbm, idx_v, rows_v, sem):
        wid = jax.lax.axis_index("s") * info.num_cores + jax.lax.axis_index("c")
        base = wid * bpw
        for c in range(nchunk):
            off = base + c * chunk
            pltpu.sync_copy(dest_hbm.at[pl.ds(off, chunk)], idx_v)
            pltpu.async_copy(ys_hbm.at[idx_v], rows_v, sem).wait()
            pltpu.sync_copy(rows_v, out_hbm.at[pl.ds(off, chunk)])

    return k(ys, dest)


# speedup vs baseline: 1.2156x; 1.0696x over previous
import functools

import jax
import jax.numpy as jnp
from jax.experimental import pallas as pl
from jax.experimental.pallas import tpu as pltpu
from jax.experimental.pallas import tpu_sc as plsc

ALPHA = 0.05
BLK = 512
KH = 2


def _router_kernel(x_ref, wg_ref, bg_ref, idx_ref, p_ref, cnt_ref, aux_ref):
    x = x_ref[...]
    wg = wg_ref[...]
    e = wg.shape[0]
    logits = jax.lax.dot_general(
        x, wg, (((1,), (1,)), ((), ())), preferred_element_type=jnp.float32
    ) + bg_ref[...]
    lmax = jnp.max(logits, axis=1, keepdims=True)
    el = jnp.exp(logits - lmax)
    sum_el = jnp.sum(el, axis=1, keepdims=True)
    probs = el / sum_el
    iota = jax.lax.broadcasted_iota(jnp.int32, logits.shape, 1)
    idx = jnp.min(jnp.where(logits == lmax, iota, e), axis=1, keepdims=True)
    idx_ref[...] = idx
    p_ref[...] = 1.0 / sum_el
    onehot = (iota == idx).astype(jnp.float32)
    cnt = jnp.sum(onehot, axis=0, keepdims=True)
    cnt_ref[...] = cnt
    ce = jnp.mean(probs, axis=0, keepdims=True)
    me = cnt / x.shape[0]
    aux_ref[...] = jnp.reshape((ALPHA * e) * jnp.sum(me * ce), (1, 1))


def _up_kernel(be_ref, fl_ref, xs_ref, wu_ref, wv_ref, hs_ref):
    b = pl.program_id(1)

    @pl.when(fl_ref[b] == 1)
    def _():
        xb = xs_ref[...]
        ut = jax.lax.dot_general(
            wu_ref[0], xb, (((1,), (1,)), ((), ())),
            preferred_element_type=jnp.float32)
        vt = jax.lax.dot_general(
            wv_ref[0], xb, (((1,), (1,)), ((), ())),
            preferred_element_type=jnp.float32)
        hs_ref[...] = ((ut * jax.nn.sigmoid(ut)) * vt).astype(jnp.bfloat16)


def _down_kernel(be_ref, fl_ref, hs_ref, ps_ref, wd_ref, ys_ref):
    b = pl.program_id(0)

    @pl.when(fl_ref[b] == 1)
    def _():
        ht_ = hs_ref[...].astype(jnp.float32)
        y = jax.lax.dot_general(
            ht_, wd_ref[0], (((0,), (1,)), ((), ())),
            preferred_element_type=jnp.float32)
        ys_ref[...] = y * ps_ref[...]

    @pl.when(fl_ref[b] == 0)
    def _():
        ys_ref[...] = jnp.zeros_like(ys_ref)


@functools.partial(jax.jit, static_argnames=())
def kernel(x, Wg, bg, Wu, Wv, Wd):
    n, d = x.shape
    e, h, _ = Wu.shape
    ht = h // KH
    nb = n // BLK + e
    s = nb * BLK

    idx2, p2, cnt2, aux2 = pl.pallas_call(
        _router_kernel,
        out_shape=[
            jax.ShapeDtypeStruct((n, 1), jnp.int32),
            jax.ShapeDtypeStruct((n, 1), jnp.float32),
            jax.ShapeDtypeStruct((1, e), jnp.float32),
            jax.ShapeDtypeStruct((1, 1), jnp.float32),
        ],
    )(x, Wg, bg.reshape(1, e))
    top1_idx = idx2[:, 0]
    top1_p = p2[:, 0]
    counts = cnt2[0].astype(jnp.int32)
    aux = aux2.reshape(())

    nblk_e = (counts + BLK - 1) // BLK
    cum_nblk = jnp.cumsum(nblk_e)
    pstart = (cum_nblk - nblk_e) * BLK
    nba = cum_nblk[-1]

    onehot = jax.nn.one_hot(top1_idx, e, dtype=jnp.int32)
    rank = jnp.take_along_axis(jnp.cumsum(onehot, axis=0),
                               top1_idx[:, None], axis=1)[:, 0] - 1
    dest = pstart[top1_idx] + rank

    blk_ids = jnp.arange(nb, dtype=jnp.int32)
    be = jnp.searchsorted(cum_nblk, blk_ids, side="right").astype(jnp.int32)
    active = be < e
    last_e = jnp.searchsorted(cum_nblk, nba - 1, side="right").astype(jnp.int32)
    be_safe = jnp.where(active, be, last_e)
    flags = active.astype(jnp.int32)

    xs = _sc_dispatch(x, dest, s)
    ps = jnp.zeros((s, 1), jnp.float32).at[dest, 0].set(top1_p)

    wu4 = Wu.reshape(e * KH, ht, d)
    wv4 = Wv.reshape(e * KH, ht, d)

    up_spec = pltpu.PrefetchScalarGridSpec(
        num_scalar_prefetch=2,
        grid=(KH, nb),
        in_specs=[
            pl.BlockSpec((BLK, d), lambda hi, b, be, fl: (b, 0)),
            pl.BlockSpec((1, ht, d), lambda hi, b, be, fl: (be[b] * KH + hi, 0, 0)),
            pl.BlockSpec((1, ht, d), lambda hi, b, be, fl: (be[b] * KH + hi, 0, 0)),
        ],
        out_specs=pl.BlockSpec((ht, BLK), lambda hi, b, be, fl: (hi, b)),
    )
    hs = pl.pallas_call(
        _up_kernel,
        grid_spec=up_spec,
        out_shape=jax.ShapeDtypeStruct((h, s), jnp.bfloat16),
        compiler_params=pltpu.CompilerParams(
            dimension_semantics=("arbitrary", "arbitrary"),
        ),
    )(be_safe, flags, xs, wu4, wv4)

    down_spec = pltpu.PrefetchScalarGridSpec(
        num_scalar_prefetch=2,
        grid=(nb,),
        in_specs=[
            pl.BlockSpec((h, BLK), lambda b, be, fl: (0, b)),
            pl.BlockSpec((BLK, 1), lambda b, be, fl: (b, 0)),
            pl.BlockSpec((1, d, h), lambda b, be, fl: (be[b], 0, 0)),
        ],
        out_specs=pl.BlockSpec((BLK, d), lambda b, be, fl: (b, 0)),
    )
    ys = pl.pallas_call(
        _down_kernel,
        grid_spec=down_spec,
        out_shape=jax.ShapeDtypeStruct((s, d), jnp.float32),
        compiler_params=pltpu.CompilerParams(
            dimension_semantics=("arbitrary",),
        ),
    )(be_safe, flags, hs, ps, Wd)

    y = _sc_combine(ys, dest, n, d)
    return y, aux


def _sc_dispatch(x, dest, s_):
    n, d = x.shape
    info = plsc.get_sparse_core_info()
    nw = info.num_cores * info.num_subcores
    bpw = n // nw
    chunk = 64
    nchunk = bpw // chunk
    mesh = plsc.VectorSubcoreMesh(core_axis_name="c", subcore_axis_name="s")

    @functools.partial(
        pl.kernel, mesh=mesh,
        out_type=jax.ShapeDtypeStruct((s_, d), jnp.float32),
        scratch_types=[
            pltpu.VMEM((chunk,), jnp.int32),
            pltpu.VMEM((chunk, d), jnp.float32),
            pltpu.SemaphoreType.DMA,
        ],
    )
    def k(x_hbm, dest_hbm, xs_hbm, idx_v, rows_v, sem):
        wid = jax.lax.axis_index("s") * info.num_cores + jax.lax.axis_index("c")
        base = wid * bpw
        for c in range(nchunk):
            off = base + c * chunk
            pltpu.sync_copy(x_hbm.at[pl.ds(off, chunk)], rows_v)
            pltpu.sync_copy(dest_hbm.at[pl.ds(off, chunk)], idx_v)
            pltpu.async_copy(rows_v, xs_hbm.at[idx_v], sem).wait()

    return k(x, dest)


def _sc_combine(ys, dest, n, d):
    info = plsc.get_sparse_core_info()
    nw = info.num_cores * info.num_subcores
    bpw = n // nw
    chunk = 64
    nchunk = bpw // chunk
    mesh = plsc.VectorSubcoreMesh(core_axis_name="c", subcore_axis_name="s")

    @functools.partial(
        pl.kernel, mesh=mesh,
        out_type=jax.ShapeDtypeStruct((n, d), jnp.float32),
        scratch_types=[
            pltpu.VMEM((chunk,), jnp.int32),
            pltpu.VMEM((chunk, d), jnp.float32),
            pltpu.SemaphoreType.DMA,
        ],
    )
    def k(ys_hbm, dest_hbm, out_hbm, idx_v, rows_v, sem):
        wid = jax.lax.axis_index("s") * info.num_cores + jax.lax.axis_index("c")
        base = wid * bpw
        for c in range(nchunk):
            off = base + c * chunk
            pltpu.sync_copy(dest_hbm.at[pl.ds(off, chunk)], idx_v)
            pltpu.async_copy(ys_hbm.at[idx_v], rows_v, sem).wait()
            pltpu.sync_copy(rows_v, out_hbm.at[pl.ds(off, chunk)])

    return k(ys, dest)
